# baseline (speedup 1.0000x reference)
import functools

import jax
import jax.numpy as jnp
from jax import lax
from jax.experimental import pallas as pl
from jax.experimental.pallas import tpu as pltpu
from jax.experimental.pallas import tpu_sc as plsc

N = 10000
D = 128
E = 320000
C = 40

NC = 2
NS = 16
NW = NC * NS
EPW = E // NW
CH = 40
EPWP = 10240
NCH = EPWP // CH
GRPC = 16
NPIECE = NCH // GRPC
NB = 4
ICH = 40
NPAD = 10240
RPT = NPAD // NS


def _sc_agg_body(x_hbm, src_hbm, dst_hbm, idn_hbm, zrow_hbm, pout,
                 acc, sidx, didx, iidx, rb0, rb1, rb2, rb3,
                 gs0, gs1, gs2, gs3, ss0, ss1, ss2, ss3):
    c = lax.axis_index("c")
    s = lax.axis_index("s")
    wid = c * NS + s
    i32 = jnp.int32
    base = s * RPT
    rb = [rb0, rb1, rb2, rb3]
    gsem = [gs0, gs1, gs2, gs3]
    ssem = [ss0, ss1, ss2, ss3]

    pltpu.sync_copy(zrow_hbm, rb0)
    pltpu.sync_copy(idn_hbm.at[s], iidx)

    def zchunk(k, carry):
        pltpu.async_copy(rb0, acc.at[iidx.at[k]], ss0).wait()
        return carry
    lax.fori_loop(i32(0), i32(RPT // ICH), zchunk, i32(0), unroll=False)
    plsc.subcore_barrier()

    def piece(p, carry):
        pltpu.sync_copy(src_hbm.at[wid, p], sidx)
        pltpu.sync_copy(dst_hbm.at[wid, p], didx)

        gd = [None] * GRPC
        sd = [None] * GRPC
        for i in range(GRPC):
            b = i % NB
            if i >= NB:
                sd[i - NB].wait()
            gd[i] = pltpu.async_copy(x_hbm.at[sidx.at[i32(i)]], rb[b],
                                     gsem[b])
            if i >= NB - 1:
                j = i - (NB - 1)
                gd[j].wait()
                sd[j] = pltpu.async_copy(rb[j % NB], acc.at[didx.at[i32(j)]],
                                         ssem[j % NB], add=True)
        for j in range(GRPC - (NB - 1), GRPC):
            gd[j].wait()
            sd[j] = pltpu.async_copy(rb[j % NB], acc.at[didx.at[i32(j)]],
                                     ssem[j % NB], add=True)
        for j in range(GRPC - NB, GRPC):
            sd[j].wait()
        return carry
    lax.fori_loop(i32(0), i32(NPIECE), piece, i32(0), unroll=False)

    plsc.subcore_barrier()

    def wchunk(k, carry):
        pltpu.async_copy(acc.at[iidx.at[k]], rb0, gs0).wait()
        pltpu.sync_copy(rb0, pout.at[c, pl.ds(base + k * ICH, ICH)])
        return carry
    lax.fori_loop(i32(0), i32(RPT // ICH), wchunk, i32(0), unroll=False)


def _sc_aggregate(x, src4, dst4, idn, zrow):
    mesh = plsc.VectorSubcoreMesh(core_axis_name="c", subcore_axis_name="s",
                                  num_cores=NC, num_subcores=NS)
    f = pl.kernel(
        _sc_agg_body,
        out_type=jax.ShapeDtypeStruct((NC, NPAD, D), jnp.float32),
        mesh=mesh,
        scratch_types=[
            pltpu.VMEM_SHARED((NPAD, D), jnp.float32),
            pltpu.VMEM((GRPC, CH), jnp.int32),
            pltpu.VMEM((GRPC, CH), jnp.int32),
            pltpu.VMEM((RPT // ICH, ICH), jnp.int32),
            pltpu.VMEM((CH, D), jnp.float32),
            pltpu.VMEM((CH, D), jnp.float32),
            pltpu.VMEM((CH, D), jnp.float32),
            pltpu.VMEM((CH, D), jnp.float32),
            pltpu.SemaphoreType.DMA,
            pltpu.SemaphoreType.DMA,
            pltpu.SemaphoreType.DMA,
            pltpu.SemaphoreType.DMA,
            pltpu.SemaphoreType.DMA,
            pltpu.SemaphoreType.DMA,
            pltpu.SemaphoreType.DMA,
            pltpu.SemaphoreType.DMA,
        ],
        name="sc_sage_aggregate",
    )
    return f(x, src4, dst4, idn, zrow)


def _tc_prep_body(f_ref, u1_ref, u2_ref, t_ref):
    f = f_ref[...]
    fu1 = jnp.dot(f, u1_ref[...], preferred_element_type=jnp.float32)
    fu2 = jnp.dot(f, u2_ref[...], preferred_element_type=jnp.float32)
    ones = jnp.ones((f.shape[0], 1), jnp.float32)
    zpad = jnp.zeros((f.shape[0], D - 2 * C - 1), jnp.float32)
    t_ref[...] = jnp.concatenate([fu1, fu2, ones, zpad], axis=1)


def _tc_prep(f, u1, u2):
    return pl.pallas_call(
        _tc_prep_body,
        out_shape=jax.ShapeDtypeStruct((N, D), jnp.float32),
    )(f, u1, u2)


def _tc_mid_body(p_ref, t2_ref, g1u1_ref, dd_ref):
    ssum = p_ref[0] + p_ref[1]
    deg = ssum[:, 2 * C]
    rdeg = 1.0 / jnp.maximum(deg, 1.0)
    dd_ref[...] = jnp.stack([deg, rdeg], axis=0)
    g1u1_ref[...] = ssum[:, :C] * rdeg[:, None]
    g1u2 = ssum[:, C:2 * C] * rdeg[:, None]
    zpad = jnp.zeros((ssum.shape[0], D - C), jnp.float32)
    t2_ref[...] = jnp.concatenate([g1u2, zpad], axis=1)


def _tc_mid(p):
    return pl.pallas_call(
        _tc_mid_body,
        out_shape=[jax.ShapeDtypeStruct((NPAD, D), jnp.float32),
                   jax.ShapeDtypeStruct((NPAD, C), jnp.float32),
                   jax.ShapeDtypeStruct((2, NPAD), jnp.float32)],
    )(p)


def _tc_final_body(f_ref, g1u1_ref, p2_ref, dd_ref, u0_ref, cb_ref, out_ref):
    deg = dd_ref[0]
    rdeg = dd_ref[1]
    a1 = deg * rdeg
    g2u2 = (p2_ref[0, :, :C] + p2_ref[1, :, :C]) * rdeg[:, None]
    acc = jnp.dot(f_ref[...], u0_ref[...], preferred_element_type=jnp.float32)
    out_ref[...] = (acc + g1u1_ref[...] + g2u2
                    + cb_ref[0][None, :] + a1[:, None] * cb_ref[1][None, :])


def _tc_final(f_pad, g1u1, p2, dd, u0, cb):
    return pl.pallas_call(
        _tc_final_body,
        out_shape=jax.ShapeDtypeStruct((NPAD, C), jnp.float32),
    )(f_pad, g1u1, p2, dd, u0, cb)


def kernel(features, edge_index, W_self1, W_neigh1, b1, W_self2, W_neigh2, b2):
    features = features.astype(jnp.float32)
    npad_e = EPWP - EPW
    srcw = edge_index[0].astype(jnp.int32).reshape(NW, EPW)
    dstw = edge_index[1].astype(jnp.int32).reshape(NW, EPW)
    lane = jnp.arange(npad_e, dtype=jnp.int32)
    wk = jnp.arange(NW, dtype=jnp.int32)[:, None]
    pad_src = (wk * 311 + lane[None, :] * 41) % N
    pad_dst = jnp.broadcast_to(N + (lane % (NPAD - N))[None, :], (NW, npad_e))
    src3 = jnp.concatenate([srcw, pad_src], axis=1).reshape(
        NW, NPIECE, GRPC, CH)
    dst3 = jnp.concatenate([dstw, pad_dst], axis=1).reshape(
        NW, NPIECE, GRPC, CH)

    out_dtype = jnp.result_type(features.dtype, W_self1.dtype)
    u0 = (W_self1 @ W_self2).astype(jnp.float32)
    u1 = (W_neigh1 @ W_self2 + W_self1 @ W_neigh2).astype(jnp.float32)
    u2 = (W_neigh1 @ W_neigh2).astype(jnp.float32)
    cb = jnp.stack([b1 @ W_self2 + b2, b1 @ W_neigh2]).astype(jnp.float32)

    idn = jnp.arange(NPAD, dtype=jnp.int32).reshape(NS, RPT // CH, CH)
    zrow = jnp.zeros((CH, D), jnp.float32)

    t1 = _tc_prep(features, u1, u2)
    p1 = _sc_aggregate(t1, src3, dst3, idn, zrow)
    t2, g1u1, dd = _tc_mid(p1)
    p2 = _sc_aggregate(t2, src3, dst3, idn, zrow)

    f_pad = jnp.pad(features, ((0, NPAD - N), (0, 0)))
    out = _tc_final(f_pad, g1u1, p2, dd, u0, cb)
    return out[:N].astype(out_dtype)


# speedup vs baseline: 1.0198x; 1.0198x over previous
import functools

import jax
import jax.numpy as jnp
from jax import lax
from jax.experimental import pallas as pl
from jax.experimental.pallas import tpu as pltpu
from jax.experimental.pallas import tpu_sc as plsc

N = 10000
D = 128
E = 320000
C = 40

NC = 2
NS = 16
NW = NC * NS
EPW = E // NW
CH = 80
EPWP = 10240
NCH = EPWP // CH
GRPC = 16
NPIECE = NCH // GRPC
NPAD = 10240
RPT = NPAD // NS


def _sc_agg_body(x_hbm, src_hbm, dst_hbm, idn_hbm, zrow_hbm, pout,
                 acc, sidx, didx, iidx, rbufA, rbufB,
                 gsemA, gsemB, ssemA, ssemB):
    c = lax.axis_index("c")
    s = lax.axis_index("s")
    wid = c * NS + s
    i32 = jnp.int32
    base = s * RPT

    pltpu.sync_copy(zrow_hbm, rbufA)
    pltpu.sync_copy(idn_hbm.at[s], iidx)

    zd = [pltpu.async_copy(rbufA, acc.at[iidx.at[i32(k)]], ssemA)
          for k in range(RPT // CH)]
    for d in zd:
        d.wait()
    plsc.subcore_barrier()

    def piece(p, carry):
        pltpu.sync_copy(src_hbm.at[wid, p], sidx)
        pltpu.sync_copy(dst_hbm.at[wid, p], didx)

        slot = lambda i: (rbufA, gsemA, ssemA) if i % 2 == 0 else \
            (rbufB, gsemB, ssemB)
        gd = [None] * GRPC
        sd = [None] * GRPC
        for i in range(GRPC):
            buf, gs, ss = slot(i)
            if i >= 2:
                sd[i - 2].wait()
            gd[i] = pltpu.async_copy(x_hbm.at[sidx.at[i32(i)]], buf, gs)
            if i >= 1:
                pbuf, _, pss = slot(i - 1)
                gd[i - 1].wait()
                sd[i - 1] = pltpu.async_copy(
                    pbuf, acc.at[didx.at[i32(i - 1)]], pss, add=True)
        lbuf, _, lss = slot(GRPC - 1)
        gd[GRPC - 1].wait()
        sd[GRPC - 1] = pltpu.async_copy(
            lbuf, acc.at[didx.at[i32(GRPC - 1)]], lss, add=True)
        sd[GRPC - 2].wait()
        sd[GRPC - 1].wait()
        return carry
    lax.fori_loop(i32(0), i32(NPIECE), piece, i32(0), unroll=False)

    plsc.subcore_barrier()

    nw_ch = RPT // CH
    wb = [rbufA, rbufB]
    wg = [gsemA, gsemB]
    wd = [None] * nw_ch
    for k in range(nw_ch):
        wd[k] = pltpu.async_copy(acc.at[iidx.at[i32(k)]], wb[k % 2], wg[k % 2])
        if k >= 1:
            wd[k - 1].wait()
            pltpu.sync_copy(wb[(k - 1) % 2],
                            pout.at[c, pl.ds(base + (k - 1) * CH, CH)])
    wd[nw_ch - 1].wait()
    pltpu.sync_copy(wb[(nw_ch - 1) % 2],
                    pout.at[c, pl.ds(base + (nw_ch - 1) * CH, CH)])


def _sc_aggregate(x, src4, dst4, idn, zrow):
    mesh = plsc.VectorSubcoreMesh(core_axis_name="c", subcore_axis_name="s",
                                  num_cores=NC, num_subcores=NS)
    f = pl.kernel(
        _sc_agg_body,
        out_type=jax.ShapeDtypeStruct((NC, NPAD, D), jnp.float32),
        mesh=mesh,
        scratch_types=[
            pltpu.VMEM_SHARED((NPAD, D), jnp.float32),
            pltpu.VMEM((GRPC, CH), jnp.int32),
            pltpu.VMEM((GRPC, CH), jnp.int32),
            pltpu.VMEM((RPT // CH, CH), jnp.int32),
            pltpu.VMEM((CH, D), jnp.float32),
            pltpu.VMEM((CH, D), jnp.float32),
            pltpu.SemaphoreType.DMA,
            pltpu.SemaphoreType.DMA,
            pltpu.SemaphoreType.DMA,
            pltpu.SemaphoreType.DMA,
        ],
        name="sc_sage_aggregate",
    )
    return f(x, src4, dst4, idn, zrow)


def _tc_prep_body(f_ref, u1_ref, u2_ref, t_ref):
    f = f_ref[...]
    fu1 = jnp.dot(f, u1_ref[...], preferred_element_type=jnp.float32)
    fu2 = jnp.dot(f, u2_ref[...], preferred_element_type=jnp.float32)
    ones = jnp.ones((f.shape[0], 1), jnp.float32)
    zpad = jnp.zeros((f.shape[0], D - 2 * C - 1), jnp.float32)
    t_ref[...] = jnp.concatenate([fu1, fu2, ones, zpad], axis=1)


def _tc_prep(f, u1, u2):
    return pl.pallas_call(
        _tc_prep_body,
        out_shape=jax.ShapeDtypeStruct((N, D), jnp.float32),
    )(f, u1, u2)


def _tc_mid_body(p_ref, t2_ref, g1u1_ref, dd_ref):
    ssum = p_ref[0] + p_ref[1]
    deg = ssum[:, 2 * C]
    rdeg = 1.0 / jnp.maximum(deg, 1.0)
    dd_ref[...] = jnp.stack([deg, rdeg], axis=0)
    g1u1_ref[...] = ssum[:, :C] * rdeg[:, None]
    g1u2 = ssum[:, C:2 * C] * rdeg[:, None]
    zpad = jnp.zeros((ssum.shape[0], D - C), jnp.float32)
    t2_ref[...] = jnp.concatenate([g1u2, zpad], axis=1)


def _tc_mid(p):
    return pl.pallas_call(
        _tc_mid_body,
        out_shape=[jax.ShapeDtypeStruct((NPAD, D), jnp.float32),
                   jax.ShapeDtypeStruct((NPAD, C), jnp.float32),
                   jax.ShapeDtypeStruct((2, NPAD), jnp.float32)],
    )(p)


def _tc_final_body(f_ref, g1u1_ref, p2_ref, dd_ref, u0_ref, cb_ref, out_ref):
    deg = dd_ref[0]
    rdeg = dd_ref[1]
    a1 = deg * rdeg
    g2u2 = (p2_ref[0, :, :C] + p2_ref[1, :, :C]) * rdeg[:, None]
    acc = jnp.dot(f_ref[...], u0_ref[...], preferred_element_type=jnp.float32)
    out_ref[...] = (acc + g1u1_ref[...] + g2u2
                    + cb_ref[0][None, :] + a1[:, None] * cb_ref[1][None, :])


def _tc_final(f_pad, g1u1, p2, dd, u0, cb):
    return pl.pallas_call(
        _tc_final_body,
        out_shape=jax.ShapeDtypeStruct((NPAD, C), jnp.float32),
    )(f_pad, g1u1, p2, dd, u0, cb)


def kernel(features, edge_index, W_self1, W_neigh1, b1, W_self2, W_neigh2, b2):
    features = features.astype(jnp.float32)
    npad_e = EPWP - EPW
    srcw = edge_index[0].astype(jnp.int32).reshape(NW, EPW)
    dstw = edge_index[1].astype(jnp.int32).reshape(NW, EPW)
    lane = jnp.arange(npad_e, dtype=jnp.int32)
    wk = jnp.arange(NW, dtype=jnp.int32)[:, None]
    pad_src = (wk * 311 + lane[None, :] * 41) % N
    pad_dst = jnp.broadcast_to(N + (lane % (NPAD - N))[None, :], (NW, npad_e))
    src3 = jnp.concatenate([srcw, pad_src], axis=1).reshape(
        NW, NPIECE, GRPC, CH)
    dst3 = jnp.concatenate([dstw, pad_dst], axis=1).reshape(
        NW, NPIECE, GRPC, CH)

    out_dtype = jnp.result_type(features.dtype, W_self1.dtype)
    u0 = (W_self1 @ W_self2).astype(jnp.float32)
    u1 = (W_neigh1 @ W_self2 + W_self1 @ W_neigh2).astype(jnp.float32)
    u2 = (W_neigh1 @ W_neigh2).astype(jnp.float32)
    cb = jnp.stack([b1 @ W_self2 + b2, b1 @ W_neigh2]).astype(jnp.float32)

    idn = jnp.arange(NPAD, dtype=jnp.int32).reshape(NS, RPT // CH, CH)
    zrow = jnp.zeros((CH, D), jnp.float32)

    t1 = _tc_prep(features, u1, u2)
    p1 = _sc_aggregate(t1, src3, dst3, idn, zrow)
    t2, g1u1, dd = _tc_mid(p1)
    p2 = _sc_aggregate(t2, src3, dst3, idn, zrow)

    f_pad = jnp.pad(features, ((0, NPAD - N), (0, 0)))
    out = _tc_final(f_pad, g1u1, p2, dd, u0, cb)
    return out[:N].astype(out_dtype)
